# Initial kernel scaffold; baseline (speedup 1.0000x reference)
import jax
import jax.numpy as jnp
from jax.experimental import pallas as pl


def kernel(query, key, value, edge_index, edge_attr, Wq, bq, Wk, bk, Wv, bv, Wo, bo, We, be):
    raise NotImplementedError("write your pallas kernel here")


# speedup vs baseline: 8.1802x; 8.1802x over previous
import functools
import math

import jax
import jax.numpy as jnp
from jax import lax
from jax.experimental import pallas as pl
from jax.experimental.pallas import tpu as pltpu
from jax.experimental.pallas import tpu_sc as plsc

N = 10000
E = 160000
D = 256
H = 8
HP = 16
DH = 32
INV_SCALE = 1.0 / math.sqrt(DH)
NEG = -1e30

CH = 128
NCH = E // CH
NW = 32
ROWS = 1000
NPAD = 10240
NPW = NPAD // 16
NHALF = NPAD // 2
NHPW = NHALF // 16
NTRASH = 16


def _proj_body(xq, xk, xv, wq, bq, wk, bk, wv, bv, q_o, k_o, vlo_o, vhi_o):
    q = jnp.dot(xq[...], wq[...], preferred_element_type=jnp.float32) + bq[...]
    k = jnp.dot(xk[...], wk[...], preferred_element_type=jnp.float32) + bk[...]
    v = jnp.dot(xv[...], wv[...], preferred_element_type=jnp.float32) + bv[...]
    q_o[...] = q
    k_o[...] = k
    vlo_o[...] = v[:, :128]
    vhi_o[...] = v[:, 128:]


def _scores_body(qd, ks, ea, we, be, s_o):
    r = lax.broadcasted_iota(jnp.int32, (D, HP), 0) // DH
    c = lax.broadcasted_iota(jnp.int32, (D, HP), 1)
    bd = (r == c).astype(jnp.float32)
    p = qd[...] * ks[...]
    s = jnp.dot(p, bd, preferred_element_type=jnp.float32) * INV_SCALE
    s = s + jnp.dot(ea[...], we[...], preferred_element_type=jnp.float32) + be[...]
    s_o[...] = s


def _max_body(s, m_o):
    i = pl.program_id(0)
    blk = jnp.max(s[...], axis=0, keepdims=True)

    @pl.when(i == 0)
    def _():
        m_o[...] = blk

    @pl.when(i > 0)
    def _():
        m_o[...] = jnp.maximum(m_o[...], blk)


def _expand_body(s, m, exst_o):
    ex = jnp.exp(s[...] - m[...])
    hh = lax.broadcasted_iota(jnp.int32, (HP, 128), 0)
    ch = lax.broadcasted_iota(jnp.int32, (HP, 128), 1) // DH
    elo = (hh == ch).astype(jnp.float32)
    ehi = (hh == ch + 4).astype(jnp.float32)
    exst_o[0] = jnp.dot(ex, elo, preferred_element_type=jnp.float32)
    exst_o[1] = jnp.dot(ex, ehi, preferred_element_type=jnp.float32)


def _outproj_body(lo, hi, p, wo, bo, o):
    acc = jnp.dot(lo[...] / (p[0] + 1e-16), wo[:128, :],
                  preferred_element_type=jnp.float32)
    acc = acc + jnp.dot(hi[...] / (p[1] + 1e-16), wo[128:, :],
                        preferred_element_type=jnp.float32)
    o[...] = acc + bo[...]


_MESH = plsc.VectorSubcoreMesh(core_axis_name="c", subcore_axis_name="s")


def _gather_qk_body(q_hbm, k_hbm, dst_hbm, src_hbm, qd_hbm, ks_hbm,
                    didx, sidx, qrows, krows, sem, sem2):
    c = lax.axis_index("c")
    s = lax.axis_index("s")
    w = s * 2 + c
    nloop = (NCH - w + NW - 1) // NW

    def body(i, _):
        base = (w + i * NW) * CH
        pltpu.sync_copy(dst_hbm.at[pl.ds(base, CH)], didx)
        pltpu.async_copy(q_hbm.at[didx], qrows, sem).wait()
        pltpu.sync_copy(qrows, qd_hbm.at[pl.ds(base, CH)])
        pltpu.sync_copy(src_hbm.at[pl.ds(base, CH)], sidx)
        pltpu.async_copy(k_hbm.at[sidx], krows, sem2).wait()
        pltpu.sync_copy(krows, ks_hbm.at[pl.ds(base, CH)])
        return 0

    lax.fori_loop(0, nloop, body, 0, unroll=False)


def _segsum_body(exst_hbm, dst_hbm, zeros_hbm, part_hbm,
                 didx, didx2, excbuf, accum, sem):
    c = lax.axis_index("c")
    s = lax.axis_index("s")
    cE = c * E
    trash = NHALF + s
    nloop = (NCH - s + 15) // 16

    for p in range(2):
        plsc.subcore_barrier()

        @pl.when(s == 0)
        def _():
            pltpu.sync_copy(zeros_hbm, accum)

        plsc.subcore_barrier()
        nbase = p * NHALF

        def body(i, _):
            base = (s + i * 16) * CH
            pltpu.sync_copy(dst_hbm.at[pl.ds(base, CH)], didx)
            for j in range(CH // 16):
                lo = didx[pl.ds(j * 16, 16)] - nbase
                ok = (lo >= 0) & (lo < NHALF)
                didx2[pl.ds(j * 16, 16)] = jnp.where(ok, lo, trash)
            pltpu.sync_copy(exst_hbm.at[pl.ds(cE + base, CH)], excbuf)
            pltpu.sync_copy(excbuf, accum.at[didx2], add=True)
            return 0

        lax.fori_loop(0, nloop, body, 0, unroll=False)
        plsc.subcore_barrier()
        pltpu.sync_copy(
            accum.at[pl.ds(s * NHPW, NHPW)],
            part_hbm.at[pl.ds(c * NPAD + nbase + s * NHPW, NHPW)])


def _msg_body(v2_hbm, exst_hbm, dst_hbm, src_hbm, zeros_hbm, out_hbm,
              didx, didx2, sidx, excbuf, vsbuf, accum, sem):
    c = lax.axis_index("c")
    s = lax.axis_index("s")
    cN = c * N
    cE = c * E
    trash = NHALF + s
    nloop = (NCH - s + 15) // 16

    for p in range(2):
        plsc.subcore_barrier()

        @pl.when(s == 0)
        def _():
            pltpu.sync_copy(zeros_hbm, accum)

        plsc.subcore_barrier()
        nbase = p * NHALF

        def body(i, _):
            base = (s + i * 16) * CH
            pltpu.sync_copy(dst_hbm.at[pl.ds(base, CH)], didx)
            pltpu.sync_copy(src_hbm.at[pl.ds(base, CH)], sidx)

            for j in range(CH // 16):
                sidx[pl.ds(j * 16, 16)] = sidx[pl.ds(j * 16, 16)] + cN
                lo = didx[pl.ds(j * 16, 16)] - nbase
                ok = (lo >= 0) & (lo < NHALF)
                didx2[pl.ds(j * 16, 16)] = jnp.where(ok, lo, trash)

            pltpu.sync_copy(exst_hbm.at[pl.ds(cE + base, CH)], excbuf)
            pltpu.async_copy(v2_hbm.at[sidx], vsbuf, sem).wait()

            def erow(e, _c):
                for j in range(8):
                    sl = pl.ds(16 * j, 16)
                    vsbuf[e, sl] = vsbuf[e, sl] * excbuf[e, sl]
                return 0

            lax.fori_loop(0, CH, erow, 0, unroll=2)
            pltpu.sync_copy(vsbuf, accum.at[didx2], add=True)
            return 0

        lax.fori_loop(0, nloop, body, 0, unroll=False)
        plsc.subcore_barrier()
        pltpu.sync_copy(
            accum.at[pl.ds(s * NHPW, NHPW)],
            out_hbm.at[pl.ds(c * NPAD + nbase + s * NHPW, NHPW)])


def kernel(query, key, value, edge_index, edge_attr,
           Wq, bq, Wk, bk, Wv, bv, Wo, bo, We, be):
    f32 = jnp.float32
    src = edge_index[0]
    dst = edge_index[1]
    bq2 = bq.reshape(1, D)
    bk2 = bk.reshape(1, D)
    bv2 = bv.reshape(1, D)
    bo2 = bo.reshape(1, D)
    We16 = jnp.concatenate([We, jnp.zeros((D, HP - H), f32)], axis=1)
    be16 = jnp.concatenate([be, jnp.full((HP - H,), NEG, f32)]).reshape(1, HP)

    gridN = N // ROWS
    gridE = E // ROWS

    Q, K, Vlo, Vhi = pl.pallas_call(
        _proj_body,
        grid=(gridN,),
        in_specs=[
            pl.BlockSpec((ROWS, D), lambda i: (i, 0)),
            pl.BlockSpec((ROWS, D), lambda i: (i, 0)),
            pl.BlockSpec((ROWS, D), lambda i: (i, 0)),
            pl.BlockSpec((D, D), lambda i: (0, 0)),
            pl.BlockSpec((1, D), lambda i: (0, 0)),
            pl.BlockSpec((D, D), lambda i: (0, 0)),
            pl.BlockSpec((1, D), lambda i: (0, 0)),
            pl.BlockSpec((D, D), lambda i: (0, 0)),
            pl.BlockSpec((1, D), lambda i: (0, 0)),
        ],
        out_specs=[
            pl.BlockSpec((ROWS, D), lambda i: (i, 0)),
            pl.BlockSpec((ROWS, D), lambda i: (i, 0)),
            pl.BlockSpec((ROWS, 128), lambda i: (i, 0)),
            pl.BlockSpec((ROWS, 128), lambda i: (i, 0)),
        ],
        out_shape=[
            jax.ShapeDtypeStruct((N, D), f32),
            jax.ShapeDtypeStruct((N, D), f32),
            jax.ShapeDtypeStruct((N, 128), f32),
            jax.ShapeDtypeStruct((N, 128), f32),
        ],
    )(query, key, value, Wq, bq2, Wk, bk2, Wv, bv2)

    V2 = jnp.concatenate([Vlo, Vhi], axis=0)

    gather_qk = functools.partial(
        pl.kernel,
        mesh=_MESH,
        out_type=[
            jax.ShapeDtypeStruct((E, D), f32),
            jax.ShapeDtypeStruct((E, D), f32),
        ],
        scratch_types=[
            pltpu.VMEM((CH,), jnp.int32),
            pltpu.VMEM((CH,), jnp.int32),
            pltpu.VMEM((CH, D), f32),
            pltpu.VMEM((CH, D), f32),
            pltpu.SemaphoreType.DMA,
            pltpu.SemaphoreType.DMA,
        ],
    )(_gather_qk_body)
    QD, KS = gather_qk(Q, K, dst, src)

    S = pl.pallas_call(
        _scores_body,
        grid=(gridE,),
        in_specs=[
            pl.BlockSpec((ROWS, D), lambda i: (i, 0)),
            pl.BlockSpec((ROWS, D), lambda i: (i, 0)),
            pl.BlockSpec((ROWS, D), lambda i: (i, 0)),
            pl.BlockSpec((D, HP), lambda i: (0, 0)),
            pl.BlockSpec((1, HP), lambda i: (0, 0)),
        ],
        out_specs=pl.BlockSpec((ROWS, HP), lambda i: (i, 0)),
        out_shape=jax.ShapeDtypeStruct((E, HP), f32),
    )(QD, KS, edge_attr, We16, be16)

    M = pl.pallas_call(
        _max_body,
        grid=(gridE,),
        in_specs=[pl.BlockSpec((ROWS, HP), lambda i: (i, 0))],
        out_specs=pl.BlockSpec((1, HP), lambda i: (0, 0)),
        out_shape=jax.ShapeDtypeStruct((1, HP), f32),
    )(S)

    EXST = pl.pallas_call(
        _expand_body,
        grid=(gridE,),
        in_specs=[
            pl.BlockSpec((ROWS, HP), lambda i: (i, 0)),
            pl.BlockSpec((1, HP), lambda i: (0, 0)),
        ],
        out_specs=pl.BlockSpec((2, ROWS, 128), lambda i: (0, i, 0)),
        out_shape=jax.ShapeDtypeStruct((2, E, 128), f32),
    )(S, M)
    EXF = EXST.reshape(2 * E, 128)

    zerosH128 = jnp.zeros((NHALF + NTRASH, 128), f32)

    segsum = functools.partial(
        pl.kernel,
        mesh=_MESH,
        out_type=jax.ShapeDtypeStruct((2 * NPAD, 128), f32),
        scratch_types=[
            pltpu.VMEM((CH,), jnp.int32),
            pltpu.VMEM((CH,), jnp.int32),
            pltpu.VMEM((CH, 128), f32),
            pltpu.VMEM_SHARED((NHALF + NTRASH, 128), f32),
            pltpu.SemaphoreType.DMA,
        ],
    )(_segsum_body)
    PART = segsum(EXF, dst, zerosH128)

    msg = functools.partial(
        pl.kernel,
        mesh=_MESH,
        out_type=jax.ShapeDtypeStruct((2 * NPAD, 128), f32),
        scratch_types=[
            pltpu.VMEM((CH,), jnp.int32),
            pltpu.VMEM((CH,), jnp.int32),
            pltpu.VMEM((CH,), jnp.int32),
            pltpu.VMEM((CH, 128), f32),
            pltpu.VMEM((CH, 128), f32),
            pltpu.VMEM_SHARED((NHALF + NTRASH, 128), f32),
            pltpu.SemaphoreType.DMA,
        ],
    )(_msg_body)
    OUTP = msg(V2, EXF, dst, src, zerosH128)

    out = pl.pallas_call(
        _outproj_body,
        grid=(gridN,),
        in_specs=[
            pl.BlockSpec((ROWS, 128), lambda i: (i, 0)),
            pl.BlockSpec((ROWS, 128), lambda i: (i, 0)),
            pl.BlockSpec((2, ROWS, 128), lambda i: (0, i, 0)),
            pl.BlockSpec((D, D), lambda i: (0, 0)),
            pl.BlockSpec((1, D), lambda i: (0, 0)),
        ],
        out_specs=pl.BlockSpec((ROWS, D), lambda i: (i, 0)),
        out_shape=jax.ShapeDtypeStruct((N, D), f32),
    )(OUTP[:N], OUTP[NPAD:NPAD + N], PART.reshape(2, NPAD, 128), Wo, bo2)

    return out


# speedup vs baseline: 12.2639x; 1.4992x over previous
import functools
import math

import jax
import jax.numpy as jnp
from jax import lax
from jax.experimental import pallas as pl
from jax.experimental.pallas import tpu as pltpu
from jax.experimental.pallas import tpu_sc as plsc

N = 10000
E = 160000
D = 256
H = 8
HP = 16
DH = 32
INV_SCALE = 1.0 / math.sqrt(DH)
NEG = -1e30

CH = 128
NCH = E // CH
NW = 32
ROWS = 1000
NPAD = 10240
NPW = NPAD // 16
NHALF = NPAD // 2
NHPW = NHALF // 16
NTRASH = 16


def _proj_body(xq, xk, xv, wq, bq, wk, bk, wv, bv, q_o, k_o, vlo_o, vhi_o):
    q = jnp.dot(xq[...], wq[...], preferred_element_type=jnp.float32) + bq[...]
    k = jnp.dot(xk[...], wk[...], preferred_element_type=jnp.float32) + bk[...]
    v = jnp.dot(xv[...], wv[...], preferred_element_type=jnp.float32) + bv[...]
    q_o[...] = q
    k_o[...] = k
    vlo_o[...] = v[:, :128]
    vhi_o[...] = v[:, 128:]


def _scores_body(qd, ks, ea, we, be, s_o):
    r = lax.broadcasted_iota(jnp.int32, (D, HP), 0) // DH
    c = lax.broadcasted_iota(jnp.int32, (D, HP), 1)
    bd = (r == c).astype(jnp.float32)
    p = qd[...] * ks[...]
    s = jnp.dot(p, bd, preferred_element_type=jnp.float32) * INV_SCALE
    s = s + jnp.dot(ea[...], we[...], preferred_element_type=jnp.float32) + be[...]
    s_o[...] = s


def _max_body(s, m_o):
    i = pl.program_id(0)
    blk = jnp.max(s[...], axis=0, keepdims=True)

    @pl.when(i == 0)
    def _():
        m_o[...] = blk

    @pl.when(i > 0)
    def _():
        m_o[...] = jnp.maximum(m_o[...], blk)


def _expand_body(s, m, exst_o):
    ex = jnp.exp(s[...] - m[...])
    hh = lax.broadcasted_iota(jnp.int32, (HP, 128), 0)
    ch = lax.broadcasted_iota(jnp.int32, (HP, 128), 1) // DH
    elo = (hh == ch).astype(jnp.float32)
    ehi = (hh == ch + 4).astype(jnp.float32)
    exst_o[0] = jnp.dot(ex, elo, preferred_element_type=jnp.float32)
    exst_o[1] = jnp.dot(ex, ehi, preferred_element_type=jnp.float32)


def _outproj_body(lo, hi, p, wo, bo, o):
    acc = jnp.dot(lo[...] / (p[0] + 1e-16), wo[:128, :],
                  preferred_element_type=jnp.float32)
    acc = acc + jnp.dot(hi[...] / (p[1] + 1e-16), wo[128:, :],
                        preferred_element_type=jnp.float32)
    o[...] = acc + bo[...]


_MESH = plsc.VectorSubcoreMesh(core_axis_name="c", subcore_axis_name="s")


def _gather_qk_body(q_hbm, k_hbm, dst_hbm, src_hbm, qd_hbm, ks_hbm,
                    didx, sidx, qrows, krows, sem, sem2):
    c = lax.axis_index("c")
    s = lax.axis_index("s")
    w = s * 2 + c
    nloop = (NCH - w + NW - 1) // NW

    def body(i, _):
        base = (w + i * NW) * CH
        pltpu.sync_copy(dst_hbm.at[pl.ds(base, CH)], didx)
        pltpu.async_copy(q_hbm.at[didx], qrows, sem).wait()
        pltpu.sync_copy(qrows, qd_hbm.at[pl.ds(base, CH)])
        pltpu.sync_copy(src_hbm.at[pl.ds(base, CH)], sidx)
        pltpu.async_copy(k_hbm.at[sidx], krows, sem2).wait()
        pltpu.sync_copy(krows, ks_hbm.at[pl.ds(base, CH)])
        return 0

    lax.fori_loop(0, nloop, body, 0, unroll=False)


def _segsum_body(exst_hbm, dst_hbm, zeros_hbm, part_hbm,
                 didx, excbuf, accum, sem):
    c = lax.axis_index("c")
    s = lax.axis_index("s")
    cE = c * E
    nloop = (NCH - s + 15) // 16

    @pl.when(s == 0)
    def _():
        pltpu.sync_copy(zeros_hbm, accum)

    plsc.subcore_barrier()

    def body(i, _):
        base = (s + i * 16) * CH
        pltpu.sync_copy(dst_hbm.at[pl.ds(base, CH)], didx)
        pltpu.sync_copy(exst_hbm.at[pl.ds(cE + base, CH)], excbuf)
        pltpu.sync_copy(excbuf, accum.at[didx], add=True)
        return 0

    lax.fori_loop(0, nloop, body, 0, unroll=False)
    plsc.subcore_barrier()
    pltpu.sync_copy(accum.at[pl.ds(s * NPW, NPW)],
                    part_hbm.at[pl.ds(c * NPAD + s * NPW, NPW)])


def _msg_body(v2_hbm, exst_hbm, dst_hbm, src_hbm, zeros_hbm, out_hbm,
              didx, sidx, excbuf, vsbuf, accum, sem):
    c = lax.axis_index("c")
    s = lax.axis_index("s")
    cN = c * N
    cE = c * E
    nloop = (NCH - s + 15) // 16

    @pl.when(s == 0)
    def _():
        pltpu.sync_copy(zeros_hbm, accum)

    plsc.subcore_barrier()

    def body(i, _):
        base = (s + i * 16) * CH
        pltpu.sync_copy(dst_hbm.at[pl.ds(base, CH)], didx)
        pltpu.sync_copy(src_hbm.at[pl.ds(base, CH)], sidx)

        for j in range(CH // 16):
            sidx[pl.ds(j * 16, 16)] = sidx[pl.ds(j * 16, 16)] + cN

        pltpu.sync_copy(exst_hbm.at[pl.ds(cE + base, CH)], excbuf)
        pltpu.async_copy(v2_hbm.at[sidx], vsbuf, sem).wait()

        def erow(e, _c):
            for j in range(8):
                sl = pl.ds(16 * j, 16)
                vsbuf[e, sl] = vsbuf[e, sl] * excbuf[e, sl]
            return 0

        lax.fori_loop(0, CH, erow, 0, unroll=2)
        pltpu.sync_copy(vsbuf, accum.at[didx], add=True)
        return 0

    lax.fori_loop(0, nloop, body, 0, unroll=False)
    plsc.subcore_barrier()
    pltpu.sync_copy(accum.at[pl.ds(s * NPW, NPW)],
                    out_hbm.at[pl.ds(c * NPAD + s * NPW, NPW)])


def kernel(query, key, value, edge_index, edge_attr,
           Wq, bq, Wk, bk, Wv, bv, Wo, bo, We, be):
    f32 = jnp.float32
    src = edge_index[0]
    dst = edge_index[1]
    bq2 = bq.reshape(1, D)
    bk2 = bk.reshape(1, D)
    bv2 = bv.reshape(1, D)
    bo2 = bo.reshape(1, D)
    We16 = jnp.concatenate([We, jnp.zeros((D, HP - H), f32)], axis=1)
    be16 = jnp.concatenate([be, jnp.full((HP - H,), NEG, f32)]).reshape(1, HP)

    gridN = N // ROWS
    gridE = E // ROWS

    Q, K, Vlo, Vhi = pl.pallas_call(
        _proj_body,
        grid=(gridN,),
        in_specs=[
            pl.BlockSpec((ROWS, D), lambda i: (i, 0)),
            pl.BlockSpec((ROWS, D), lambda i: (i, 0)),
            pl.BlockSpec((ROWS, D), lambda i: (i, 0)),
            pl.BlockSpec((D, D), lambda i: (0, 0)),
            pl.BlockSpec((1, D), lambda i: (0, 0)),
            pl.BlockSpec((D, D), lambda i: (0, 0)),
            pl.BlockSpec((1, D), lambda i: (0, 0)),
            pl.BlockSpec((D, D), lambda i: (0, 0)),
            pl.BlockSpec((1, D), lambda i: (0, 0)),
        ],
        out_specs=[
            pl.BlockSpec((ROWS, D), lambda i: (i, 0)),
            pl.BlockSpec((ROWS, D), lambda i: (i, 0)),
            pl.BlockSpec((ROWS, 128), lambda i: (i, 0)),
            pl.BlockSpec((ROWS, 128), lambda i: (i, 0)),
        ],
        out_shape=[
            jax.ShapeDtypeStruct((N, D), f32),
            jax.ShapeDtypeStruct((N, D), f32),
            jax.ShapeDtypeStruct((N, 128), f32),
            jax.ShapeDtypeStruct((N, 128), f32),
        ],
    )(query, key, value, Wq, bq2, Wk, bk2, Wv, bv2)

    V2 = jnp.concatenate([Vlo, Vhi], axis=0)

    gather_qk = functools.partial(
        pl.kernel,
        mesh=_MESH,
        out_type=[
            jax.ShapeDtypeStruct((E, D), f32),
            jax.ShapeDtypeStruct((E, D), f32),
        ],
        scratch_types=[
            pltpu.VMEM((CH,), jnp.int32),
            pltpu.VMEM((CH,), jnp.int32),
            pltpu.VMEM((CH, D), f32),
            pltpu.VMEM((CH, D), f32),
            pltpu.SemaphoreType.DMA,
            pltpu.SemaphoreType.DMA,
        ],
    )(_gather_qk_body)
    QD, KS = gather_qk(Q, K, dst, src)

    S = pl.pallas_call(
        _scores_body,
        grid=(gridE,),
        in_specs=[
            pl.BlockSpec((ROWS, D), lambda i: (i, 0)),
            pl.BlockSpec((ROWS, D), lambda i: (i, 0)),
            pl.BlockSpec((ROWS, D), lambda i: (i, 0)),
            pl.BlockSpec((D, HP), lambda i: (0, 0)),
            pl.BlockSpec((1, HP), lambda i: (0, 0)),
        ],
        out_specs=pl.BlockSpec((ROWS, HP), lambda i: (i, 0)),
        out_shape=jax.ShapeDtypeStruct((E, HP), f32),
    )(QD, KS, edge_attr, We16, be16)

    M = pl.pallas_call(
        _max_body,
        grid=(gridE,),
        in_specs=[pl.BlockSpec((ROWS, HP), lambda i: (i, 0))],
        out_specs=pl.BlockSpec((1, HP), lambda i: (0, 0)),
        out_shape=jax.ShapeDtypeStruct((1, HP), f32),
    )(S)

    EXST = pl.pallas_call(
        _expand_body,
        grid=(gridE,),
        in_specs=[
            pl.BlockSpec((ROWS, HP), lambda i: (i, 0)),
            pl.BlockSpec((1, HP), lambda i: (0, 0)),
        ],
        out_specs=pl.BlockSpec((2, ROWS, 128), lambda i: (0, i, 0)),
        out_shape=jax.ShapeDtypeStruct((2, E, 128), f32),
    )(S, M)
    EXF = EXST.reshape(2 * E, 128)

    zerosF128 = jnp.zeros((NPAD, 128), f32)

    segsum = functools.partial(
        pl.kernel,
        mesh=_MESH,
        out_type=jax.ShapeDtypeStruct((2 * NPAD, 128), f32),
        scratch_types=[
            pltpu.VMEM((CH,), jnp.int32),
            pltpu.VMEM((CH, 128), f32),
            pltpu.VMEM_SHARED((NPAD, 128), f32),
            pltpu.SemaphoreType.DMA,
        ],
    )(_segsum_body)
    PART = segsum(EXF, dst, zerosF128)

    msg = functools.partial(
        pl.kernel,
        mesh=_MESH,
        out_type=jax.ShapeDtypeStruct((2 * NPAD, 128), f32),
        scratch_types=[
            pltpu.VMEM((CH,), jnp.int32),
            pltpu.VMEM((CH,), jnp.int32),
            pltpu.VMEM((CH, 128), f32),
            pltpu.VMEM((CH, 128), f32),
            pltpu.VMEM_SHARED((NPAD, 128), f32),
            pltpu.SemaphoreType.DMA,
        ],
    )(_msg_body)
    OUTP = msg(V2, EXF, dst, src, zerosF128)

    out = pl.pallas_call(
        _outproj_body,
        grid=(gridN,),
        in_specs=[
            pl.BlockSpec((ROWS, 128), lambda i: (i, 0)),
            pl.BlockSpec((ROWS, 128), lambda i: (i, 0)),
            pl.BlockSpec((2, ROWS, 128), lambda i: (0, i, 0)),
            pl.BlockSpec((D, D), lambda i: (0, 0)),
            pl.BlockSpec((1, D), lambda i: (0, 0)),
        ],
        out_specs=pl.BlockSpec((ROWS, D), lambda i: (i, 0)),
        out_shape=jax.ShapeDtypeStruct((N, D), f32),
    )(OUTP[:N], OUTP[NPAD:NPAD + N], PART.reshape(2, NPAD, 128), Wo, bo2)

    return out
